# baseline (speedup 1.0000x reference)
import jax
import jax.numpy as jnp
from jax import lax
from jax.experimental import pallas as pl
from jax.experimental.pallas import tpu as pltpu
from jax.experimental.pallas import tpu_sc as plsc

_E = 8
_TOPK = 2
_H = 768
_R = 16
_B = 64
_S = 512

_NC = 2
_NS = 16
_NW = _NC * _NS
_SPW = _B // _NW
_G = 64
_CPS = _S // _G
_NCHUNK = _SPW * _CPS
_NBUF = 2
_LANES = 16
_COLV = _H // _LANES
_KEEP = 32
_SCAT = _G - _KEEP


def _bag_body(table, ids3, out, ids_v, buf0, buf1, acc_v, sem0, sem1):
    cid = lax.axis_index("c")
    sid = lax.axis_index("s")
    wid = sid * _NC + cid

    pltpu.sync_copy(ids3.at[wid], ids_v)

    bufs = (buf0, buf1)
    sems = (sem0, sem1)

    def issue(k):
        return pltpu.async_copy(
            table.at[ids_v.at[k]], bufs[k % _NBUF], sems[k % _NBUF]
        )

    def accumulate(k):
        samp = k // _CPS
        first = (k % _CPS) == 0
        buf = bufs[k % _NBUF]

        def col_body(j, carry):
            for half in range(2):
                base = (2 * j + half) * _LANES
                parts = [buf[r, pl.ds(base, _LANES)] for r in range(4)]
                for r in range(4, _G):
                    parts[r % 4] = parts[r % 4] + buf[r, pl.ds(base, _LANES)]
                s = (parts[0] + parts[1]) + (parts[2] + parts[3])
                if first:
                    acc_v[samp, pl.ds(base, _LANES)] = s
                else:
                    acc_v[samp, pl.ds(base, _LANES)] = (
                        acc_v[samp, pl.ds(base, _LANES)] + s
                    )
            return carry

        lax.fori_loop(0, _COLV // 2, col_body, 0)

    inflight = [issue(k) for k in range(_NBUF - 1)]
    for k in range(_NCHUNK):
        if k + _NBUF - 1 < _NCHUNK:
            inflight.append(issue(k + _NBUF - 1))
        inflight.pop(0).wait()
        accumulate(k)

    for samp in range(_SPW):
        pltpu.sync_copy(acc_v.at[samp], out.at[wid * _SPW + samp])


@jax.jit
def _embedding_bag(emb_table, ids3):
    mesh = plsc.VectorSubcoreMesh(
        core_axis_name="c", subcore_axis_name="s", num_cores=_NC, num_subcores=_NS
    )
    return pl.kernel(
        _bag_body,
        out_type=jax.ShapeDtypeStruct((_B, _H), jnp.float32),
        mesh=mesh,
        scratch_types=[
            pltpu.VMEM((_NCHUNK, _G), jnp.int32),
            pltpu.VMEM((_G, _H), jnp.float32),
            pltpu.VMEM((_G, _H), jnp.float32),
            pltpu.VMEM((_SPW, _H), jnp.float32),
            pltpu.SemaphoreType.DMA,
            pltpu.SemaphoreType.DMA,
        ],
    )(emb_table, ids3)


def _combine_body(x_ref, w_ref, b_ref, sums_ref, la_ref, lb_ref,
                  pred_ref, probs_ref, load_ref):
    x = x_ref[...]
    logits = (
        lax.dot_general(
            x, w_ref[...], (((1,), (1,)), ((), ())),
            preferred_element_type=jnp.float32,
        )
        + b_ref[...]
    )

    it = lax.broadcasted_iota(jnp.int32, (_B, _E), 1)
    m1 = jnp.max(logits, axis=1, keepdims=True)
    a1 = jnp.min(jnp.where(logits == m1, it, _E), axis=1, keepdims=True)
    masked = jnp.where(it == a1, -jnp.inf, logits)
    m2 = jnp.max(masked, axis=1, keepdims=True)
    a2 = jnp.min(jnp.where(masked == m2, it, _E), axis=1, keepdims=True)

    w1 = 1.0 / (1.0 + jnp.exp(m2 - m1))
    w2 = 1.0 - w1

    ex = jnp.exp(logits - m1)
    probs_ref[...] = ex / jnp.sum(ex, axis=1, keepdims=True)

    oh = jnp.where(it == a1, 1.0, 0.0) + jnp.where(it == a2, 1.0, 0.0)
    load_ref[...] = jnp.sum(oh, axis=0, keepdims=True)

    M = sums_ref[...] * (1.0 / _S)

    eb = lax.broadcasted_iota(jnp.int32, (_B, _E * _R), 1) // _R
    c = jnp.where(eb == a1, w1, 0.0) + jnp.where(eb == a2, w2, 0.0)

    t = jnp.dot(M, la_ref[...], preferred_element_type=jnp.float32)
    delta = jnp.dot(t * c, lb_ref[...], preferred_element_type=jnp.float32)
    pred_ref[...] = M + delta


@jax.jit
def _combine(router_inputs, router_W, router_b2, sums, lora_A, lora_B):
    return pl.pallas_call(
        _combine_body,
        out_shape=(
            jax.ShapeDtypeStruct((_B, _H), jnp.float32),
            jax.ShapeDtypeStruct((_B, _E), jnp.float32),
            jax.ShapeDtypeStruct((1, _E), jnp.float32),
        ),
    )(router_inputs, router_W, router_b2, sums, lora_A, lora_B)


def kernel(router_inputs, input_ids, router_W, router_b, emb_table, lora_A, lora_B):
    ids3 = input_ids.reshape(_NW, _NCHUNK, _G)
    sums = _embedding_bag(emb_table, ids3)

    a_cat = lora_A.transpose(1, 0, 2).reshape(_H, _E * _R)
    b_cat = lora_B.reshape(_E * _R, _H)
    pred, probs, load = _combine(
        router_inputs, router_W, router_b.reshape(1, _E), sums, a_cat, b_cat
    )
    return pred, probs, load.reshape(_E)


# speedup vs baseline: 1.2121x; 1.2121x over previous
import jax
import jax.numpy as jnp
from jax import lax
from jax.experimental import pallas as pl
from jax.experimental.pallas import tpu as pltpu
from jax.experimental.pallas import tpu_sc as plsc

_E = 8
_TOPK = 2
_H = 768
_R = 16
_B = 64
_S = 512

_NC = 2
_NS = 16
_NW = _NC * _NS
_SPW = _B // _NW
_G = 64
_CPS = _S // _G
_NCHUNK = _SPW * _CPS
_NBUF = 2
_LANES = 16
_COLV = _H // _LANES
_GRP = 12
_NGRP = _COLV // _GRP


def _bag_body(table, ids3, out, ids_v, buf0, buf1, acc_v, sem0, sem1):
    cid = lax.axis_index("c")
    sid = lax.axis_index("s")
    wid = sid * _NC + cid

    pltpu.sync_copy(ids3.at[wid], ids_v)

    bufs = (buf0, buf1)
    sems = (sem0, sem1)

    def issue(k):
        return pltpu.async_copy(
            table.at[ids_v.at[k]], bufs[k % _NBUF], sems[k % _NBUF]
        )

    zero = jnp.zeros((_LANES,), jnp.float32)
    for samp in range(_SPW):
        for j in range(_COLV):
            acc_v[samp, pl.ds(j * _LANES, _LANES)] = zero

    def accumulate(k):
        samp = k // _CPS
        buf = bufs[k % _NBUF]
        for g in range(_NGRP):
            base = g * _GRP * _LANES
            init = tuple(
                acc_v[samp, pl.ds(base + j * _LANES, _LANES)]
                for j in range(_GRP)
            )

            def row_add(r, carry):
                return tuple(
                    carry[j] + buf[r, pl.ds(base + j * _LANES, _LANES)]
                    for j in range(_GRP)
                )

            res = lax.fori_loop(0, _G, row_add, init)
            for j in range(_GRP):
                acc_v[samp, pl.ds(base + j * _LANES, _LANES)] = res[j]

    inflight = [issue(k) for k in range(_NBUF - 1)]
    for k in range(_NCHUNK):
        if k + _NBUF - 1 < _NCHUNK:
            inflight.append(issue(k + _NBUF - 1))
        inflight.pop(0).wait()
        accumulate(k)

    for samp in range(_SPW):
        pltpu.sync_copy(acc_v.at[samp], out.at[wid * _SPW + samp])


@jax.jit
def _embedding_bag(emb_table, ids3):
    mesh = plsc.VectorSubcoreMesh(
        core_axis_name="c", subcore_axis_name="s", num_cores=_NC, num_subcores=_NS
    )
    return pl.kernel(
        _bag_body,
        out_type=jax.ShapeDtypeStruct((_B, _H), jnp.float32),
        mesh=mesh,
        scratch_types=[
            pltpu.VMEM((_NCHUNK, _G), jnp.int32),
            pltpu.VMEM((_G, _H), jnp.float32),
            pltpu.VMEM((_G, _H), jnp.float32),
            pltpu.VMEM((_SPW, _H), jnp.float32),
            pltpu.SemaphoreType.DMA,
            pltpu.SemaphoreType.DMA,
        ],
    )(emb_table, ids3)


def _combine_body(x_ref, w_ref, b_ref, sums_ref, la_ref, lb_ref,
                  pred_ref, probs_ref, load_ref):
    x = x_ref[...]
    logits = (
        lax.dot_general(
            x, w_ref[...], (((1,), (1,)), ((), ())),
            preferred_element_type=jnp.float32,
        )
        + b_ref[...]
    )

    it = lax.broadcasted_iota(jnp.int32, (_B, _E), 1)
    m1 = jnp.max(logits, axis=1, keepdims=True)
    a1 = jnp.min(jnp.where(logits == m1, it, _E), axis=1, keepdims=True)
    masked = jnp.where(it == a1, -jnp.inf, logits)
    m2 = jnp.max(masked, axis=1, keepdims=True)
    a2 = jnp.min(jnp.where(masked == m2, it, _E), axis=1, keepdims=True)

    w1 = 1.0 / (1.0 + jnp.exp(m2 - m1))
    w2 = 1.0 - w1

    ex = jnp.exp(logits - m1)
    probs_ref[...] = ex / jnp.sum(ex, axis=1, keepdims=True)

    oh = jnp.where(it == a1, 1.0, 0.0) + jnp.where(it == a2, 1.0, 0.0)
    load_ref[...] = jnp.sum(oh, axis=0, keepdims=True)

    M = sums_ref[...] * (1.0 / _S)

    eb = lax.broadcasted_iota(jnp.int32, (_B, _E * _R), 1) // _R
    c = jnp.where(eb == a1, w1, 0.0) + jnp.where(eb == a2, w2, 0.0)

    t = jnp.dot(M, la_ref[...], preferred_element_type=jnp.float32)
    delta = jnp.dot(t * c, lb_ref[...], preferred_element_type=jnp.float32)
    pred_ref[...] = M + delta


@jax.jit
def _combine(router_inputs, router_W, router_b2, sums, lora_A, lora_B):
    return pl.pallas_call(
        _combine_body,
        out_shape=(
            jax.ShapeDtypeStruct((_B, _H), jnp.float32),
            jax.ShapeDtypeStruct((_B, _E), jnp.float32),
            jax.ShapeDtypeStruct((1, _E), jnp.float32),
        ),
    )(router_inputs, router_W, router_b2, sums, lora_A, lora_B)


def kernel(router_inputs, input_ids, router_W, router_b, emb_table, lora_A, lora_B):
    ids3 = input_ids.reshape(_NW, _NCHUNK, _G)
    sums = _embedding_bag(emb_table, ids3)

    a_cat = lora_A.transpose(1, 0, 2).reshape(_H, _E * _R)
    b_cat = lora_B.reshape(_E * _R, _H)
    pred, probs, load = _combine(
        router_inputs, router_W, router_b.reshape(1, _E), sums, a_cat, b_cat
    )
    return pred, probs, load.reshape(_E)
